# baseline (speedup 1.0000x reference)
import functools
import math

import jax
import jax.numpy as jnp
from jax import lax
from jax.experimental import pallas as pl
from jax.experimental.pallas import tpu as pltpu
from jax.experimental.pallas import tpu_sc as plsc

EPS = 1e-8
NCOMP = 16
LANES = 16
NC, NS = 2, 16
NW = NC * NS
N = 1024 * 2048
PER_W = N // NW
CHUNK = 8192
NCHUNK = PER_W // CHUNK
VPC = CHUNK // LANES


def _sc_gmm(w_flat, mu_b, nis_b, coef_b, invt_b):
    mesh = plsc.VectorSubcoreMesh(core_axis_name="c", subcore_axis_name="s")

    @functools.partial(
        pl.kernel,
        mesh=mesh,
        out_type=jax.ShapeDtypeStruct((N,), jnp.float32),
        scratch_types=[
            pltpu.VMEM((NCOMP, LANES), jnp.float32),
            pltpu.VMEM((NCOMP, LANES), jnp.float32),
            pltpu.VMEM((NCOMP, LANES), jnp.float32),
            pltpu.VMEM((LANES,), jnp.float32),
            pltpu.VMEM((CHUNK,), jnp.float32),
            pltpu.VMEM((CHUNK,), jnp.float32),
        ],
    )
    def body(w_hbm, mu_hbm, nis_hbm, coef_hbm, invt_hbm, out_hbm,
             mu_v, nis_v, coef_v, invt_v, wbuf, obuf):
        wid = lax.axis_index("s") * NC + lax.axis_index("c")
        base = wid * PER_W
        pltpu.sync_copy(mu_hbm, mu_v)
        pltpu.sync_copy(nis_hbm, nis_v)
        pltpu.sync_copy(coef_hbm, coef_v)
        pltpu.sync_copy(invt_hbm, invt_v)

        @pl.loop(0, NCHUNK)
        def chunk_body(j):
            off = base + j * CHUNK
            pltpu.sync_copy(w_hbm.at[pl.ds(off, CHUNK)], wbuf)

            @plsc.parallel_loop(0, CHUNK, step=LANES, unroll=2)
            def vec_body(i):
                invt = invt_v[...]
                w = wbuf[pl.ds(i, LANES)]
                us = []
                for k in range(NCOMP):
                    d = w - mu_v[k]
                    us.append(jnp.exp(d * d * nis_v[k] + coef_v[k]))
                def tree(vals, op):
                    while len(vals) > 1:
                        vals = [op(vals[a], vals[a + 1])
                                for a in range(0, len(vals) - 1, 2)] + (
                                    [vals[-1]] if len(vals) % 2 else [])
                    return vals[0]
                s = tree(list(us), lambda a, b: a + b)
                m = tree(list(us), jnp.maximum)
                c = invt / (s + EPS)
                mc = m * c
                es = [jnp.exp(us[k] * c - mc) for k in range(NCOMP)]
                den = tree(list(es), lambda a, b: a + b)
                num = tree([es[k] * mu_v[k] for k in range(1, NCOMP)],
                           lambda a, b: a + b)
                obuf[pl.ds(i, LANES)] = num / den

            pltpu.sync_copy(obuf, out_hbm.at[pl.ds(off, CHUNK)])

    return body(w_flat, mu_b, nis_b, coef_b, invt_b)


def kernel(weights, mu, pi_k, pi_zero, sigma, sigma_zero, temperature):
    w = weights.reshape(-1)
    pi_tmp = jnp.abs(jnp.concatenate([pi_zero, pi_k]))
    pi_norm = pi_tmp / jnp.sum(pi_tmp)
    mu_all = jnp.concatenate([jnp.zeros((1,), weights.dtype), mu])
    sigma_all = jnp.concatenate([sigma_zero, sigma])
    two_sig2 = 2.0 * sigma_all ** 2
    coef = pi_norm / jnp.sqrt(math.pi * two_sig2)
    nis = -1.0 / two_sig2
    lc = jnp.log(coef)

    mu_b = jnp.broadcast_to(mu_all[:, None], (NCOMP, LANES))
    nis_b = jnp.broadcast_to(nis[:, None], (NCOMP, LANES))
    coef_b = jnp.broadcast_to(lc[:, None], (NCOMP, LANES))
    invt_b = jnp.broadcast_to(1.0 / temperature, (LANES,))

    out = _sc_gmm(w, mu_b, nis_b, coef_b, invt_b)
    return out.reshape(weights.shape)


# speedup vs baseline: 1.3601x; 1.3601x over previous
import functools
import math

import jax
import jax.numpy as jnp
from jax import lax
from jax.experimental import pallas as pl
from jax.experimental.pallas import tpu as pltpu
from jax.experimental.pallas import tpu_sc as plsc

EPS = 1e-8
NCOMP = 16
LANES = 16
NC, NS = 2, 16
NW = NC * NS
N = 1024 * 2048
PER_W = N // NW
CHUNK = 8192
NCHUNK = PER_W // CHUNK
VPC = CHUNK // LANES


def _sc_gmm(w_flat, mu_b, nis_b, coef_b, invt_b):
    mesh = plsc.VectorSubcoreMesh(core_axis_name="c", subcore_axis_name="s")

    @functools.partial(
        pl.kernel,
        mesh=mesh,
        out_type=jax.ShapeDtypeStruct((N,), jnp.float32),
        scratch_types=[
            pltpu.VMEM((NCOMP, LANES), jnp.float32),
            pltpu.VMEM((NCOMP, LANES), jnp.float32),
            pltpu.VMEM((NCOMP, LANES), jnp.float32),
            pltpu.VMEM((LANES,), jnp.float32),
            pltpu.VMEM((CHUNK,), jnp.float32),
            pltpu.VMEM((CHUNK,), jnp.float32),
        ],
    )
    def body(w_hbm, mu_hbm, nis_hbm, coef_hbm, invt_hbm, out_hbm,
             mu_v, nis_v, coef_v, invt_v, wbuf, obuf):
        wid = lax.axis_index("s") * NC + lax.axis_index("c")
        base = wid * PER_W
        pltpu.sync_copy(mu_hbm, mu_v)
        pltpu.sync_copy(nis_hbm, nis_v)
        pltpu.sync_copy(coef_hbm, coef_v)
        pltpu.sync_copy(invt_hbm, invt_v)

        @pl.loop(0, NCHUNK)
        def chunk_body(j):
            off = base + j * CHUNK
            pltpu.sync_copy(w_hbm.at[pl.ds(off, CHUNK)], wbuf)

            @plsc.parallel_loop(0, CHUNK, step=LANES, unroll=2)
            def vec_body(i):
                invt = invt_v[...]
                w = wbuf[pl.ds(i, LANES)]
                us = []
                sp = [None] * 4
                mp = [None] * 4
                for k in range(NCOMP):
                    d = w - mu_v[k]
                    u = jnp.exp(d * d * nis_v[k] + coef_v[k])
                    us.append(u)
                    a = k & 3
                    sp[a] = u if sp[a] is None else sp[a] + u
                    mp[a] = u if mp[a] is None else jnp.maximum(mp[a], u)
                s = (sp[0] + sp[1]) + (sp[2] + sp[3])
                m = jnp.maximum(jnp.maximum(mp[0], mp[1]),
                                jnp.maximum(mp[2], mp[3]))
                c = invt / (s + EPS)
                mc = m * c
                dp = [None] * 4
                np_ = [None] * 2
                for k in range(NCOMP):
                    e = jnp.exp(us[k] * c - mc)
                    a = k & 3
                    dp[a] = e if dp[a] is None else dp[a] + e
                    if k > 0:
                        b = k & 1
                        t = e * mu_v[k]
                        np_[b] = t if np_[b] is None else np_[b] + t
                den = (dp[0] + dp[1]) + (dp[2] + dp[3])
                num = np_[0] + np_[1]
                obuf[pl.ds(i, LANES)] = num / den

            pltpu.sync_copy(obuf, out_hbm.at[pl.ds(off, CHUNK)])

    return body(w_flat, mu_b, nis_b, coef_b, invt_b)


def kernel(weights, mu, pi_k, pi_zero, sigma, sigma_zero, temperature):
    w = weights.reshape(-1)
    pi_tmp = jnp.abs(jnp.concatenate([pi_zero, pi_k]))
    pi_norm = pi_tmp / jnp.sum(pi_tmp)
    mu_all = jnp.concatenate([jnp.zeros((1,), weights.dtype), mu])
    sigma_all = jnp.concatenate([sigma_zero, sigma])
    two_sig2 = 2.0 * sigma_all ** 2
    coef = pi_norm / jnp.sqrt(math.pi * two_sig2)
    nis = -1.0 / two_sig2
    lc = jnp.log(coef)

    mu_b = jnp.broadcast_to(mu_all[:, None], (NCOMP, LANES))
    nis_b = jnp.broadcast_to(nis[:, None], (NCOMP, LANES))
    coef_b = jnp.broadcast_to(lc[:, None], (NCOMP, LANES))
    invt_b = jnp.broadcast_to(1.0 / temperature, (LANES,))

    out = _sc_gmm(w, mu_b, nis_b, coef_b, invt_b)
    return out.reshape(weights.shape)


# speedup vs baseline: 1.5185x; 1.1165x over previous
import functools
import math

import jax
import jax.numpy as jnp
from jax import lax
from jax.experimental import pallas as pl
from jax.experimental.pallas import tpu as pltpu
from jax.experimental.pallas import tpu_sc as plsc

EPS = 1e-8
NCOMP = 16
LANES = 16
NC, NS = 2, 16
NW = NC * NS
N = 1024 * 2048
PER_W = N // NW
CHUNK = 8192
NCHUNK = PER_W // CHUNK
VPC = CHUNK // LANES


def _sc_gmm(w_flat, mu_b, nis_b, coef_b, invt_b):
    mesh = plsc.VectorSubcoreMesh(core_axis_name="c", subcore_axis_name="s")

    @functools.partial(
        pl.kernel,
        mesh=mesh,
        out_type=jax.ShapeDtypeStruct((N,), jnp.float32),
        scratch_types=[
            pltpu.VMEM((NCOMP, LANES), jnp.float32),
            pltpu.VMEM((NCOMP, LANES), jnp.float32),
            pltpu.VMEM((NCOMP, LANES), jnp.float32),
            pltpu.VMEM((LANES,), jnp.float32),
            pltpu.VMEM((CHUNK,), jnp.float32),
            pltpu.VMEM((CHUNK,), jnp.float32),
        ],
    )
    def body(w_hbm, mu_hbm, nis_hbm, coef_hbm, invt_hbm, out_hbm,
             mu_v, nis_v, coef_v, invt_v, wbuf, obuf):
        wid = lax.axis_index("s") * NC + lax.axis_index("c")
        base = wid * PER_W
        pltpu.sync_copy(mu_hbm, mu_v)
        pltpu.sync_copy(nis_hbm, nis_v)
        pltpu.sync_copy(coef_hbm, coef_v)
        pltpu.sync_copy(invt_hbm, invt_v)

        @pl.loop(0, NCHUNK)
        def chunk_body(j):
            off = base + j * CHUNK
            pltpu.sync_copy(w_hbm.at[pl.ds(off, CHUNK)], wbuf)

            @plsc.parallel_loop(0, CHUNK, step=LANES, unroll=2)
            def vec_body(i):
                invt = invt_v[...]
                w = wbuf[pl.ds(i, LANES)]
                us = []
                s = None
                m = None
                for k in range(NCOMP):
                    d = w - mu_v[k]
                    u = jnp.exp(d * d * nis_v[k] + coef_v[k])
                    us.append(u)
                    if k == 0:
                        s = u
                        m = u
                    else:
                        s = s + u
                        m = jnp.maximum(m, u)
                c = invt / (s + EPS)
                mc = m * c
                den = None
                num = None
                for k in range(NCOMP):
                    e = jnp.exp(us[k] * c - mc)
                    if k == 0:
                        den = e
                    else:
                        den = den + e
                        num = e * mu_v[k] if k == 1 else num + e * mu_v[k]
                obuf[pl.ds(i, LANES)] = num / den

            pltpu.sync_copy(obuf, out_hbm.at[pl.ds(off, CHUNK)])

    return body(w_flat, mu_b, nis_b, coef_b, invt_b)


def kernel(weights, mu, pi_k, pi_zero, sigma, sigma_zero, temperature):
    w = weights.reshape(-1)
    pi_tmp = jnp.abs(jnp.concatenate([pi_zero, pi_k]))
    pi_norm = pi_tmp / jnp.sum(pi_tmp)
    mu_all = jnp.concatenate([jnp.zeros((1,), weights.dtype), mu])
    sigma_all = jnp.concatenate([sigma_zero, sigma])
    two_sig2 = 2.0 * sigma_all ** 2
    coef = pi_norm / jnp.sqrt(math.pi * two_sig2)
    nis = -1.0 / two_sig2
    lc = jnp.log(coef)

    mu_b = jnp.broadcast_to(mu_all[:, None], (NCOMP, LANES))
    nis_b = jnp.broadcast_to(nis[:, None], (NCOMP, LANES))
    coef_b = jnp.broadcast_to(lc[:, None], (NCOMP, LANES))
    invt_b = jnp.broadcast_to(1.0 / temperature, (LANES,))

    out = _sc_gmm(w, mu_b, nis_b, coef_b, invt_b)
    return out.reshape(weights.shape)


# speedup vs baseline: 4.2128x; 2.7742x over previous
import functools
import math

import jax
import jax.numpy as jnp
from jax import lax
from jax.experimental import pallas as pl
from jax.experimental.pallas import tpu as pltpu
from jax.experimental.pallas import tpu_sc as plsc

EPS = 1e-8
NCOMP = 16
LANES = 16
NC, NS = 2, 16
NW = NC * NS
ROWS, COLS = 1024, 2048
R_SC = 256
R_TC = ROWS - R_SC
CHUNK = 8192
BLK = 64


def _sc_gmm(w_flat, mu_b, nis_b, coef_b, invt_b):
    n = R_SC * COLS
    per_w = n // NW
    nchunk = per_w // CHUNK
    mesh = plsc.VectorSubcoreMesh(core_axis_name="c", subcore_axis_name="s")

    @functools.partial(
        pl.kernel,
        mesh=mesh,
        out_type=jax.ShapeDtypeStruct((n,), jnp.float32),
        scratch_types=[
            pltpu.VMEM((NCOMP, LANES), jnp.float32),
            pltpu.VMEM((NCOMP, LANES), jnp.float32),
            pltpu.VMEM((NCOMP, LANES), jnp.float32),
            pltpu.VMEM((LANES,), jnp.float32),
            pltpu.VMEM((CHUNK,), jnp.float32),
            pltpu.VMEM((CHUNK,), jnp.float32),
        ],
    )
    def body(w_hbm, mu_hbm, nis_hbm, coef_hbm, invt_hbm, out_hbm,
             mu_v, nis_v, coef_v, invt_v, wbuf, obuf):
        wid = lax.axis_index("s") * NC + lax.axis_index("c")
        base = wid * per_w
        pltpu.sync_copy(mu_hbm, mu_v)
        pltpu.sync_copy(nis_hbm, nis_v)
        pltpu.sync_copy(coef_hbm, coef_v)
        pltpu.sync_copy(invt_hbm, invt_v)

        @pl.loop(0, nchunk)
        def chunk_body(j):
            off = base + j * CHUNK
            pltpu.sync_copy(w_hbm.at[pl.ds(off, CHUNK)], wbuf)

            @plsc.parallel_loop(0, CHUNK, step=LANES, unroll=2)
            def vec_body(i):
                invt = invt_v[...]
                w = wbuf[pl.ds(i, LANES)]
                us = []
                s = None
                m = None
                for k in range(NCOMP):
                    d = w - mu_v[k]
                    u = jnp.exp(d * d * nis_v[k] + coef_v[k])
                    us.append(u)
                    if k == 0:
                        s = u
                        m = u
                    else:
                        s = s + u
                        m = jnp.maximum(m, u)
                c = invt / (s + EPS)
                mc = m * c
                den = None
                num = None
                for k in range(NCOMP):
                    e = jnp.exp(us[k] * c - mc)
                    if k == 0:
                        den = e
                    else:
                        den = den + e
                        num = e * mu_v[k] if k == 1 else num + e * mu_v[k]
                obuf[pl.ds(i, LANES)] = num / den

            pltpu.sync_copy(obuf, out_hbm.at[pl.ds(off, CHUNK)])

    return body(w_flat, mu_b, nis_b, coef_b, invt_b)


def _tc_body(mu_ref, nis_ref, coef_ref, invt_ref, w_ref, o_ref):
    w = w_ref[...]
    invt = invt_ref[0]
    us = []
    s = None
    m = None
    for k in range(NCOMP):
        d = w - mu_ref[k]
        u = jnp.exp(d * d * nis_ref[k] + coef_ref[k])
        us.append(u)
        if k == 0:
            s = u
            m = u
        else:
            s = s + u
            m = jnp.maximum(m, u)
    c = invt / (s + EPS)
    mc = m * c
    den = None
    num = None
    for k in range(NCOMP):
        e = jnp.exp(us[k] * c - mc)
        if k == 0:
            den = e
        else:
            den = den + e
            num = e * mu_ref[k] if k == 1 else num + e * mu_ref[k]
    o_ref[...] = num / den


def _tc_gmm(w2d, mu_all, nis, lc, invt):
    return pl.pallas_call(
        _tc_body,
        grid=(R_TC // BLK,),
        in_specs=[
            pl.BlockSpec(memory_space=pltpu.SMEM),
            pl.BlockSpec(memory_space=pltpu.SMEM),
            pl.BlockSpec(memory_space=pltpu.SMEM),
            pl.BlockSpec(memory_space=pltpu.SMEM),
            pl.BlockSpec((BLK, COLS), lambda i: (i, 0)),
        ],
        out_specs=pl.BlockSpec((BLK, COLS), lambda i: (i, 0)),
        out_shape=jax.ShapeDtypeStruct((R_TC, COLS), jnp.float32),
    )(mu_all, nis, lc, invt, w2d)


def kernel(weights, mu, pi_k, pi_zero, sigma, sigma_zero, temperature):
    pi_tmp = jnp.abs(jnp.concatenate([pi_zero, pi_k]))
    pi_norm = pi_tmp / jnp.sum(pi_tmp)
    mu_all = jnp.concatenate([jnp.zeros((1,), weights.dtype), mu])
    sigma_all = jnp.concatenate([sigma_zero, sigma])
    two_sig2 = 2.0 * sigma_all ** 2
    coef = pi_norm / jnp.sqrt(math.pi * two_sig2)
    nis = -1.0 / two_sig2
    lc = jnp.log(coef)
    invt = 1.0 / temperature

    mu_b = jnp.broadcast_to(mu_all[:, None], (NCOMP, LANES))
    nis_b = jnp.broadcast_to(nis[:, None], (NCOMP, LANES))
    coef_b = jnp.broadcast_to(lc[:, None], (NCOMP, LANES))
    invt_b = jnp.broadcast_to(invt, (LANES,))

    w_sc = weights[:R_SC].reshape(-1)
    out_sc = _sc_gmm(w_sc, mu_b, nis_b, coef_b, invt_b)
    out_tc = _tc_gmm(weights[R_SC:], mu_all, nis, lc, invt)
    return jnp.concatenate([out_sc.reshape(R_SC, COLS), out_tc], axis=0)
